# baseline (speedup 1.0000x reference)
import functools

import numpy as np
import jax
import jax.numpy as jnp
from jax.experimental import pallas as pl
from jax.experimental.pallas import tpu as pltpu

_C = 256
_H = 64
_K = 16
_OH = 1024
_NPTS = 10
_NROWS = 12
_HIGH = jax.lax.Precision.HIGHEST


def _resize_weights(in_size: int, out_size: int) -> np.ndarray:
    inv = in_size / out_size
    sample = (np.arange(out_size, dtype=np.float64) + 0.5) * inv - 0.5
    x = np.abs(sample[None, :] - np.arange(in_size, dtype=np.float64)[:, None])
    w = np.maximum(0.0, 1.0 - x)
    w = w / w.sum(axis=0, keepdims=True)
    return w.astype(np.float32)


_WX = _resize_weights(_H, _OH)
_WY = np.ascontiguousarray(_WX.T)

_ECOLS = ([0, 23]
          + sum([[16 * m + 8, 16 * m + 23] for m in range(1, 62)], [])
          + [1000, 1023])
_ECOLS = _ECOLS + [0, 0]
_WXET = np.ascontiguousarray(_WX[:, _ECOLS].T)


def _sim_body(t_ref, r_ref, sim_ref):
    rr = r_ref[...]
    rn = rr / (jnp.sqrt(jnp.sum(rr * rr, axis=1, keepdims=True)) + 1e-6)
    rnb = rn.astype(jnp.bfloat16)
    ch = t_ref[...]
    norm = jnp.sqrt(jnp.sum(ch * ch, axis=0, keepdims=True)) + 1e-6
    tnb = (ch / norm).astype(jnp.bfloat16)
    sim_ref[...] = jax.lax.dot_general(rnb, tnb, (((1,), (0,)), ((), ())),
                                       preferred_element_type=jnp.float32)


def _dot(a, b):
    return jax.lax.dot_general(a, b, (((1,), (0,)), ((), ())),
                               preferred_element_type=jnp.float32,
                               precision=_HIGH)


def _sel_body(sim_ref, simt_ref, wx_ref, wxet_ref, wy_ref, sc_ref, ix_ref,
              x_scr, cand_scr, rm_scr, rmin_scr):
    big = jnp.int32(1 << 30)
    neg = jnp.float32(-jnp.inf)

    x_scr[...] = _dot(sim_ref[...], wx_ref[...])

    for k in range(_K):
        xet = _dot(wxet_ref[...], simt_ref[pl.ds(_H * k, _H), :])
        tilet = _dot(xet, wx_ref[...])
        rm_scr[pl.ds(k, 1), :] = jnp.max(tilet, axis=0).reshape(1, _OH)
        rmin_scr[pl.ds(k, 1), :] = jnp.min(tilet, axis=0).reshape(1, _OH)

    riota = jax.lax.broadcasted_iota(jnp.int32, (_K, _OH), 1)
    slot = jax.lax.broadcasted_iota(jnp.int32, (1, 16), 1)

    rm = rm_scr[...]
    r_sel = jnp.zeros((_K, 16), jnp.int32)
    rv0 = None
    for i in range(_NROWS):
        mv = jnp.max(rm, axis=1, keepdims=True)
        rv = jnp.min(jnp.where(rm == mv, riota, big), axis=1, keepdims=True)
        if i == 0:
            rv0 = rv
        r_sel = jnp.where(slot == i, rv, r_sel)
        rm = jnp.where(riota == rv, neg, rm)
    rmin = rmin_scr[...]
    for i in range(2):
        mnv = jnp.min(rmin, axis=1, keepdims=True)
        rb = jnp.min(jnp.where(rmin == mnv, riota, big), axis=1,
                     keepdims=True)
        r_sel = jnp.where(slot == _NROWS + i, rb, r_sel)
        rmin = jnp.where(riota == rb, -neg, rmin)
    r_sel = jnp.where(slot > _NROWS + 1, rv0, r_sel)

    col3 = jax.lax.broadcasted_iota(jnp.int32, (_K, 16, _OH), 2)
    oh3 = jnp.where(col3 == r_sel[:, :, None], jnp.float32(1.0),
                    jnp.float32(0.0))
    for kk in range(_K):
        rows_w = _dot(oh3[kk], wy_ref[...])
        cand_scr[pl.ds(16 * kk, 16), :] = _dot(
            rows_w, x_scr[pl.ds(_H * kk, _H), :])

    cand = cand_scr[...].reshape(_K, 16, _OH)
    lane = jax.lax.broadcasted_iota(jnp.int32, (1, 128), 1)

    cm = jnp.max(cand, axis=1)
    cmn = jnp.min(cand[:, _NROWS:_NROWS + 2, :], axis=1)
    c_sel = jnp.zeros((_K, 16), jnp.int32)
    cv0 = None
    for i in range(_NROWS):
        mv = jnp.max(cm, axis=1, keepdims=True)
        cv = jnp.min(jnp.where(cm == mv, riota, big), axis=1, keepdims=True)
        if i == 0:
            cv0 = cv
        c_sel = jnp.where(slot == i, cv, c_sel)
        cm = jnp.where(riota == cv, neg, cm)
    for i in range(2):
        mnv = jnp.min(cmn, axis=1, keepdims=True)
        cb = jnp.min(jnp.where(cmn == mnv, riota, big), axis=1,
                     keepdims=True)
        c_sel = jnp.where(slot == _NROWS + i, cb, c_sel)
        cmn = jnp.where(riota == cb, -neg, cmn)
    c_sel = jnp.where(slot > _NROWS + 1, cv0, c_sel)

    ohc3 = jnp.where(col3 == c_sel[:, :, None], jnp.float32(1.0),
                     jnp.float32(0.0))
    c2_list = []
    for kk in range(_K):
        c2_list.append(jax.lax.dot_general(
            cand[kk], ohc3[kk], (((1,), (1,)), ((), ())),
            preferred_element_type=jnp.float32, precision=_HIGH))
    cand2 = jnp.stack(c2_list)
    gidx2 = r_sel[:, :, None] * _OH + c_sel[:, None, :]

    bgrow = cand2[:, _NROWS:_NROWS + 2, :]
    bgg = gidx2[:, _NROWS:_NROWS + 2, :]
    mnb = jnp.min(jnp.min(bgrow, axis=2), axis=1)[:, None, None]
    gbg = jnp.min(jnp.min(jnp.where(bgrow == mnb, bgg, big), axis=2),
                  axis=1)[:, None]
    ix_mat = jnp.where(lane == _NPTS, gbg,
                       jnp.zeros((_K, 128), jnp.int32))
    sc_mat = jnp.zeros((_K, 128), jnp.float32)

    for tt in range(_NPTS):
        m2 = jnp.max(cand2, axis=2)
        m = jnp.max(m2, axis=1)[:, None, None]
        g3 = jnp.where(cand2 == m, gidx2, big)
        g2 = jnp.min(g3, axis=2)
        g = jnp.min(g2, axis=1)[:, None]
        sc_mat = jnp.where(lane == tt, m[:, :, 0], sc_mat)
        ix_mat = jnp.where(lane == tt, g, ix_mat)
        cand2 = jnp.where(gidx2 == g[:, :, None], neg, cand2)

    sc_ref[...] = sc_mat
    ix_ref[...] = ix_mat


@functools.partial(jax.jit, static_argnames=("interpret",))
def _run(target2, reference_feats, interpret=False):
    sim = pl.pallas_call(
        _sim_body,
        grid=(16,),
        in_specs=[
            pl.BlockSpec((_C, 256), lambda j: (0, j)),
            pl.BlockSpec((_K, _C), lambda j: (0, 0)),
        ],
        out_specs=pl.BlockSpec((_K, 256), lambda j: (0, j)),
        out_shape=jax.ShapeDtypeStruct((_K, _H * _H), jnp.float32),
        interpret=interpret,
    )(target2, reference_feats)

    sim2 = sim.reshape(_K * _H, _H)
    simt = sim.reshape(_K, _H, _H).transpose(0, 2, 1).reshape(_K * _H, _H)

    sc, ix = pl.pallas_call(
        _sel_body,
        out_shape=[
            jax.ShapeDtypeStruct((_K, 128), jnp.float32),
            jax.ShapeDtypeStruct((_K, 128), jnp.int32),
        ],
        scratch_shapes=[
            pltpu.VMEM((_K * _H, _OH), jnp.float32),
            pltpu.VMEM((_K * 16, _OH), jnp.float32),
            pltpu.VMEM((_K, _OH), jnp.float32),
            pltpu.VMEM((_K, _OH), jnp.float32),
        ],
        interpret=interpret,
    )(sim2, simt, jnp.asarray(_WX), jnp.asarray(_WXET), jnp.asarray(_WY))
    return sc, ix


def kernel(image_embeddings, reference_feats, orig_h, orig_w):
    target2 = image_embeddings.reshape(_C, _H * _H)
    sc, ix = _run(target2, reference_feats)
    scores = sc[:, :_NPTS]
    idx = ix[:, :_NPTS]
    xs = (idx % orig_w).astype(jnp.float32)
    ys = ((idx % (orig_h * orig_w)) // orig_w).astype(jnp.float32)
    points_scores = jnp.stack([xs, ys, scores], axis=-1)
    bgi = ix[:, _NPTS:_NPTS + 1]
    bg_x = (bgi % orig_w).astype(jnp.float32)
    bg_y = ((bgi % (orig_h * orig_w)) // orig_w).astype(jnp.float32)
    bg_coords = jnp.stack([bg_x, bg_y], axis=-1)
    return points_scores, bg_coords


# speedup vs baseline: 1.1862x; 1.1862x over previous
import functools

import numpy as np
import jax
import jax.numpy as jnp
from jax.experimental import pallas as pl
from jax.experimental.pallas import tpu as pltpu

_C = 256
_H = 64
_K = 16
_OH = 1024
_NPTS = 10
_NROWS = 12
_HIGH = jax.lax.Precision.HIGHEST


def _resize_weights(in_size: int, out_size: int) -> np.ndarray:
    inv = in_size / out_size
    sample = (np.arange(out_size, dtype=np.float64) + 0.5) * inv - 0.5
    x = np.abs(sample[None, :] - np.arange(in_size, dtype=np.float64)[:, None])
    w = np.maximum(0.0, 1.0 - x)
    w = w / w.sum(axis=0, keepdims=True)
    return w.astype(np.float32)


_WX = _resize_weights(_H, _OH)
_WY = np.ascontiguousarray(_WX.T)

_ECOLS = ([0, 23]
          + sum([[16 * m + 8, 16 * m + 23] for m in range(1, 62)], [])
          + [1000, 1023])
_ECOLS = _ECOLS + [0, 0]
_WXET = np.ascontiguousarray(_WX[:, _ECOLS].T)


def _sim_body(t_ref, r_ref, sim_ref):
    rr = r_ref[...]
    rn = rr / (jnp.sqrt(jnp.sum(rr * rr, axis=1, keepdims=True)) + 1e-6)
    rnb = rn.astype(jnp.bfloat16)
    for j in range(4):
        ch = t_ref[:, pl.ds(j * 256, 256)]
        norm = jnp.sqrt(jnp.sum(ch * ch, axis=0, keepdims=True)) + 1e-6
        tnb = (ch / norm).astype(jnp.bfloat16)
        s = jax.lax.dot_general(rnb, tnb, (((1,), (0,)), ((), ())),
                                preferred_element_type=jnp.float32)
        sim_ref[:, pl.ds(j * 256, 256)] = s


def _dot(a, b):
    return jax.lax.dot_general(a, b, (((1,), (0,)), ((), ())),
                               preferred_element_type=jnp.float32,
                               precision=_HIGH)


def _sel_body(sim_ref, simt_ref, wx_ref, wxet_ref, wy_ref, sc_ref, ix_ref,
              x_scr, cand_scr, rm_scr, rmin_scr):
    big = jnp.int32(1 << 30)
    neg = jnp.float32(-jnp.inf)

    x_scr[...] = _dot(sim_ref[...], wx_ref[...])

    for k in range(_K):
        xet = _dot(wxet_ref[...], simt_ref[pl.ds(_H * k, _H), :])
        tilet = _dot(xet, wx_ref[...])
        rm_scr[pl.ds(k, 1), :] = jnp.max(tilet, axis=0).reshape(1, _OH)
        rmin_scr[pl.ds(k, 1), :] = jnp.min(tilet, axis=0).reshape(1, _OH)

    riota = jax.lax.broadcasted_iota(jnp.int32, (_K, _OH), 1)
    slot = jax.lax.broadcasted_iota(jnp.int32, (1, 16), 1)

    rm = rm_scr[...]
    r_sel = jnp.zeros((_K, 16), jnp.int32)
    rv0 = None
    for i in range(_NROWS):
        mv = jnp.max(rm, axis=1, keepdims=True)
        rv = jnp.min(jnp.where(rm == mv, riota, big), axis=1, keepdims=True)
        if i == 0:
            rv0 = rv
        r_sel = jnp.where(slot == i, rv, r_sel)
        rm = jnp.where(riota == rv, neg, rm)
    rmin = rmin_scr[...]
    for i in range(2):
        mnv = jnp.min(rmin, axis=1, keepdims=True)
        rb = jnp.min(jnp.where(rmin == mnv, riota, big), axis=1,
                     keepdims=True)
        r_sel = jnp.where(slot == _NROWS + i, rb, r_sel)
        rmin = jnp.where(riota == rb, -neg, rmin)
    r_sel = jnp.where(slot > _NROWS + 1, rv0, r_sel)

    col3 = jax.lax.broadcasted_iota(jnp.int32, (_K, 16, _OH), 2)
    oh3 = jnp.where(col3 == r_sel[:, :, None], jnp.float32(1.0),
                    jnp.float32(0.0))
    for kk in range(_K):
        rows_w = _dot(oh3[kk], wy_ref[...])
        cand_scr[pl.ds(16 * kk, 16), :] = _dot(
            rows_w, x_scr[pl.ds(_H * kk, _H), :])

    cand = cand_scr[...].reshape(_K, 16, _OH)
    gidx = r_sel[:, :, None] * _OH + col3
    lane = jax.lax.broadcasted_iota(jnp.int32, (1, 128), 1)

    bgrow = cand[:, _NROWS:_NROWS + 2, :]
    bgg = gidx[:, _NROWS:_NROWS + 2, :]
    mnb = jnp.min(jnp.min(bgrow, axis=2), axis=1)[:, None, None]
    gbg = jnp.min(jnp.min(jnp.where(bgrow == mnb, bgg, big), axis=2),
                  axis=1)[:, None]
    ix_mat = jnp.where(lane == _NPTS, gbg,
                       jnp.zeros((_K, 128), jnp.int32))
    sc_mat = jnp.zeros((_K, 128), jnp.float32)

    for tt in range(_NPTS):
        m2 = jnp.max(cand, axis=2)
        m = jnp.max(m2, axis=1)[:, None, None]
        g3 = jnp.where(cand == m, gidx, big)
        g2 = jnp.min(g3, axis=2)
        g = jnp.min(g2, axis=1)[:, None]
        sc_mat = jnp.where(lane == tt, m[:, :, 0], sc_mat)
        ix_mat = jnp.where(lane == tt, g, ix_mat)
        cand = jnp.where(gidx == g[:, :, None], neg, cand)

    sc_ref[...] = sc_mat
    ix_ref[...] = ix_mat


@functools.partial(jax.jit, static_argnames=("interpret",))
def _run(target2, reference_feats, interpret=False):
    sim = pl.pallas_call(
        _sim_body,
        grid=(4,),
        in_specs=[
            pl.BlockSpec((_C, 1024), lambda j: (0, j)),
            pl.BlockSpec((_K, _C), lambda j: (0, 0)),
        ],
        out_specs=pl.BlockSpec((_K, 1024), lambda j: (0, j)),
        out_shape=jax.ShapeDtypeStruct((_K, _H * _H), jnp.float32),
        interpret=interpret,
    )(target2, reference_feats)

    sim2 = sim.reshape(_K * _H, _H)
    simt = sim.reshape(_K, _H, _H).transpose(0, 2, 1).reshape(_K * _H, _H)

    sc, ix = pl.pallas_call(
        _sel_body,
        out_shape=[
            jax.ShapeDtypeStruct((_K, 128), jnp.float32),
            jax.ShapeDtypeStruct((_K, 128), jnp.int32),
        ],
        scratch_shapes=[
            pltpu.VMEM((_K * _H, _OH), jnp.float32),
            pltpu.VMEM((_K * 16, _OH), jnp.float32),
            pltpu.VMEM((_K, _OH), jnp.float32),
            pltpu.VMEM((_K, _OH), jnp.float32),
        ],
        interpret=interpret,
    )(sim2, simt, jnp.asarray(_WX), jnp.asarray(_WXET), jnp.asarray(_WY))
    return sc, ix


def kernel(image_embeddings, reference_feats, orig_h, orig_w):
    target2 = image_embeddings.reshape(_C, _H * _H)
    sc, ix = _run(target2, reference_feats)
    scores = sc[:, :_NPTS]
    idx = ix[:, :_NPTS]
    xs = (idx % orig_w).astype(jnp.float32)
    ys = ((idx % (orig_h * orig_w)) // orig_w).astype(jnp.float32)
    points_scores = jnp.stack([xs, ys, scores], axis=-1)
    bgi = ix[:, _NPTS:_NPTS + 1]
    bg_x = (bgi % orig_w).astype(jnp.float32)
    bg_y = ((bgi % (orig_h * orig_w)) // orig_w).astype(jnp.float32)
    bg_coords = jnp.stack([bg_x, bg_y], axis=-1)
    return points_scores, bg_coords


# speedup vs baseline: 1.2066x; 1.0172x over previous
import functools

import numpy as np
import jax
import jax.numpy as jnp
from jax.experimental import pallas as pl
from jax.experimental.pallas import tpu as pltpu

_C = 256
_H = 64
_K = 16
_OH = 1024
_NPTS = 10
_NROWS = 12
_HIGH = jax.lax.Precision.HIGHEST


def _resize_weights(in_size: int, out_size: int) -> np.ndarray:
    inv = in_size / out_size
    sample = (np.arange(out_size, dtype=np.float64) + 0.5) * inv - 0.5
    x = np.abs(sample[None, :] - np.arange(in_size, dtype=np.float64)[:, None])
    w = np.maximum(0.0, 1.0 - x)
    w = w / w.sum(axis=0, keepdims=True)
    return w.astype(np.float32)


_WX = _resize_weights(_H, _OH)
_WY = np.ascontiguousarray(_WX.T)

_ECOLS = ([0, 23]
          + sum([[16 * m + 8, 16 * m + 23] for m in range(1, 62)], [])
          + [1000, 1023])
_ECOLS = _ECOLS + [0, 0]
_WXET = np.ascontiguousarray(_WX[:, _ECOLS].T)


def _sim_body(t_ref, r_ref, sim_ref):
    rr = r_ref[...]
    rn = rr / (jnp.sqrt(jnp.sum(rr * rr, axis=1, keepdims=True)) + 1e-6)
    rnb = rn.astype(jnp.bfloat16)
    for j in range(16):
        ch = t_ref[:, pl.ds(j * 256, 256)]
        norm = jnp.sqrt(jnp.sum(ch * ch, axis=0, keepdims=True)) + 1e-6
        tnb = (ch / norm).astype(jnp.bfloat16)
        s = jax.lax.dot_general(rnb, tnb, (((1,), (0,)), ((), ())),
                                preferred_element_type=jnp.float32)
        sim_ref[:, pl.ds(j * 256, 256)] = s


def _dot(a, b):
    return jax.lax.dot_general(a, b, (((1,), (0,)), ((), ())),
                               preferred_element_type=jnp.float32,
                               precision=_HIGH)


def _sel_body(sim_ref, simt_ref, wx_ref, wxet_ref, wy_ref, sc_ref, ix_ref,
              x_scr, cand_scr, rm_scr, rmin_scr):
    big = jnp.int32(1 << 30)
    neg = jnp.float32(-jnp.inf)

    x_scr[...] = _dot(sim_ref[...], wx_ref[...])

    for k in range(_K):
        xet = _dot(wxet_ref[...], simt_ref[pl.ds(_H * k, _H), :])
        tilet = _dot(xet, wx_ref[...])
        rm_scr[pl.ds(k, 1), :] = jnp.max(tilet, axis=0).reshape(1, _OH)
        rmin_scr[pl.ds(k, 1), :] = jnp.min(tilet, axis=0).reshape(1, _OH)

    riota = jax.lax.broadcasted_iota(jnp.int32, (_K, _OH), 1)
    slot = jax.lax.broadcasted_iota(jnp.int32, (1, 16), 1)

    rm = rm_scr[...]
    r_sel = jnp.zeros((_K, 16), jnp.int32)
    rv0 = None
    for i in range(_NROWS):
        mv = jnp.max(rm, axis=1, keepdims=True)
        rv = jnp.min(jnp.where(rm == mv, riota, big), axis=1, keepdims=True)
        if i == 0:
            rv0 = rv
        r_sel = jnp.where(slot == i, rv, r_sel)
        rm = jnp.where(riota == rv, neg, rm)
    rmin = rmin_scr[...]
    for i in range(2):
        mnv = jnp.min(rmin, axis=1, keepdims=True)
        rb = jnp.min(jnp.where(rmin == mnv, riota, big), axis=1,
                     keepdims=True)
        r_sel = jnp.where(slot == _NROWS + i, rb, r_sel)
        rmin = jnp.where(riota == rb, -neg, rmin)
    r_sel = jnp.where(slot > _NROWS + 1, rv0, r_sel)

    col3 = jax.lax.broadcasted_iota(jnp.int32, (_K, 16, _OH), 2)
    oh3 = jnp.where(col3 == r_sel[:, :, None], jnp.float32(1.0),
                    jnp.float32(0.0))
    for kk in range(_K):
        rows_w = _dot(oh3[kk], wy_ref[...])
        cand_scr[pl.ds(16 * kk, 16), :] = _dot(
            rows_w, x_scr[pl.ds(_H * kk, _H), :])

    cand = cand_scr[...].reshape(_K, 16, _OH)
    gidx = r_sel[:, :, None] * _OH + col3
    lane = jax.lax.broadcasted_iota(jnp.int32, (1, 128), 1)

    bgrow = cand[:, _NROWS:_NROWS + 2, :]
    bgg = gidx[:, _NROWS:_NROWS + 2, :]
    mnb = jnp.min(jnp.min(bgrow, axis=2), axis=1)[:, None, None]
    gbg = jnp.min(jnp.min(jnp.where(bgrow == mnb, bgg, big), axis=2),
                  axis=1)[:, None]
    ix_mat = jnp.where(lane == _NPTS, gbg,
                       jnp.zeros((_K, 128), jnp.int32))
    sc_mat = jnp.zeros((_K, 128), jnp.float32)

    for tt in range(_NPTS):
        m2 = jnp.max(cand, axis=2)
        m = jnp.max(m2, axis=1)[:, None, None]
        g3 = jnp.where(cand == m, gidx, big)
        g2 = jnp.min(g3, axis=2)
        g = jnp.min(g2, axis=1)[:, None]
        sc_mat = jnp.where(lane == tt, m[:, :, 0], sc_mat)
        ix_mat = jnp.where(lane == tt, g, ix_mat)
        cand = jnp.where(gidx == g[:, :, None], neg, cand)

    sc_ref[...] = sc_mat
    ix_ref[...] = ix_mat


@functools.partial(jax.jit, static_argnames=("interpret",))
def _run(target2, reference_feats, interpret=False):
    sim = pl.pallas_call(
        _sim_body,
        out_shape=jax.ShapeDtypeStruct((_K, _H * _H), jnp.float32),
        interpret=interpret,
    )(target2, reference_feats)

    sim2 = sim.reshape(_K * _H, _H)
    simt = sim.reshape(_K, _H, _H).transpose(0, 2, 1).reshape(_K * _H, _H)

    sc, ix = pl.pallas_call(
        _sel_body,
        out_shape=[
            jax.ShapeDtypeStruct((_K, 128), jnp.float32),
            jax.ShapeDtypeStruct((_K, 128), jnp.int32),
        ],
        scratch_shapes=[
            pltpu.VMEM((_K * _H, _OH), jnp.float32),
            pltpu.VMEM((_K * 16, _OH), jnp.float32),
            pltpu.VMEM((_K, _OH), jnp.float32),
            pltpu.VMEM((_K, _OH), jnp.float32),
        ],
        interpret=interpret,
    )(sim2, simt, jnp.asarray(_WX), jnp.asarray(_WXET), jnp.asarray(_WY))
    return sc, ix


def kernel(image_embeddings, reference_feats, orig_h, orig_w):
    target2 = image_embeddings.reshape(_C, _H * _H)
    sc, ix = _run(target2, reference_feats)
    scores = sc[:, :_NPTS]
    idx = ix[:, :_NPTS]
    xs = (idx % orig_w).astype(jnp.float32)
    ys = ((idx % (orig_h * orig_w)) // orig_w).astype(jnp.float32)
    points_scores = jnp.stack([xs, ys, scores], axis=-1)
    bgi = ix[:, _NPTS:_NPTS + 1]
    bg_x = (bgi % orig_w).astype(jnp.float32)
    bg_y = ((bgi % (orig_h * orig_w)) // orig_w).astype(jnp.float32)
    bg_coords = jnp.stack([bg_x, bg_y], axis=-1)
    return points_scores, bg_coords


# speedup vs baseline: 1.2597x; 1.0440x over previous
import functools

import numpy as np
import jax
import jax.numpy as jnp
from jax.experimental import pallas as pl
from jax.experimental.pallas import tpu as pltpu

_C = 256
_H = 64
_K = 16
_OH = 1024
_NPTS = 10
_NROWS = 12
_HIGH = jax.lax.Precision.HIGHEST


def _resize_weights(in_size: int, out_size: int) -> np.ndarray:
    inv = in_size / out_size
    sample = (np.arange(out_size, dtype=np.float64) + 0.5) * inv - 0.5
    x = np.abs(sample[None, :] - np.arange(in_size, dtype=np.float64)[:, None])
    w = np.maximum(0.0, 1.0 - x)
    w = w / w.sum(axis=0, keepdims=True)
    return w.astype(np.float32)


_WX = _resize_weights(_H, _OH)
_WY = np.ascontiguousarray(_WX.T)

_ECOLS = ([0, 23]
          + sum([[16 * m + 8, 16 * m + 23] for m in range(1, 62)], [])
          + [1000, 1023])
_ECOLS = _ECOLS + [0, 0]
_WXET = np.ascontiguousarray(_WX[:, _ECOLS].T)


def _sim_body(t_ref, r_ref, sim_ref):
    rr = r_ref[...]
    rn = rr / (jnp.sqrt(jnp.sum(rr * rr, axis=1, keepdims=True)) + 1e-6)
    rnb = rn.astype(jnp.bfloat16)
    for j in range(16):
        ch = t_ref[:, pl.ds(j * 256, 256)]
        norm = jnp.sqrt(jnp.sum(ch * ch, axis=0, keepdims=True)) + 1e-6
        tnb = (ch / norm).astype(jnp.bfloat16)
        s = jax.lax.dot_general(rnb, tnb, (((1,), (0,)), ((), ())),
                                preferred_element_type=jnp.float32)
        sim_ref[:, pl.ds(j * 256, 256)] = s


def _dot(a, b):
    return jax.lax.dot_general(a, b, (((1,), (0,)), ((), ())),
                               preferred_element_type=jnp.float32,
                               precision=_HIGH)


def _sel_body(sim_ref, simt_ref, wx_ref, wxet_ref, wy_ref, sc_ref, ix_ref,
              x_scr, cand_scr, rm_scr, rmin_scr):
    big = jnp.int32(1 << 30)
    neg = jnp.float32(-jnp.inf)

    x_scr[...] = _dot(sim_ref[...], wx_ref[...])

    for k in range(_K):
        xet = _dot(wxet_ref[...], simt_ref[pl.ds(_H * k, _H), :])
        tilet = _dot(xet, wx_ref[...])
        rm_scr[pl.ds(k, 1), :] = jnp.max(tilet, axis=0).reshape(1, _OH)
        rmin_scr[pl.ds(k, 1), :] = jnp.min(tilet, axis=0).reshape(1, _OH)

    riota = jax.lax.broadcasted_iota(jnp.int32, (_K, _OH), 1)
    slot = jax.lax.broadcasted_iota(jnp.int32, (1, 16), 1)

    rm = rm_scr[...]
    r_sel = jnp.zeros((_K, 16), jnp.int32)
    rv0 = None
    for i in range(_NROWS):
        mv = jnp.max(rm, axis=1, keepdims=True)
        rv = jnp.min(jnp.where(rm == mv, riota, big), axis=1, keepdims=True)
        if i == 0:
            rv0 = rv
        r_sel = jnp.where(slot == i, rv, r_sel)
        rm = jnp.where(riota == rv, neg, rm)
    rmin = rmin_scr[...]
    for i in range(2):
        mnv = jnp.min(rmin, axis=1, keepdims=True)
        rb = jnp.min(jnp.where(rmin == mnv, riota, big), axis=1,
                     keepdims=True)
        r_sel = jnp.where(slot == _NROWS + i, rb, r_sel)
        rmin = jnp.where(riota == rb, -neg, rmin)
    r_sel = jnp.where(slot > _NROWS + 1, rv0, r_sel)

    col3 = jax.lax.broadcasted_iota(jnp.int32, (_K, 16, _OH), 2)
    oh3 = jnp.where(col3 == r_sel[:, :, None], jnp.float32(1.0),
                    jnp.float32(0.0))
    for kk in range(_K):
        rows_w = jax.lax.dot_general(
            oh3[kk], wy_ref[...], (((1,), (0,)), ((), ())),
            preferred_element_type=jnp.float32)
        cand_scr[pl.ds(16 * kk, 16), :] = _dot(
            rows_w, x_scr[pl.ds(_H * kk, _H), :])

    cand = cand_scr[...].reshape(_K, 16, _OH)
    gidx = r_sel[:, :, None] * _OH + col3
    lane = jax.lax.broadcasted_iota(jnp.int32, (1, 128), 1)

    bgrow = cand[:, _NROWS:_NROWS + 2, :]
    bgg = gidx[:, _NROWS:_NROWS + 2, :]
    mnb = jnp.min(jnp.min(bgrow, axis=2), axis=1)[:, None, None]
    gbg = jnp.min(jnp.min(jnp.where(bgrow == mnb, bgg, big), axis=2),
                  axis=1)[:, None]
    ix_mat = jnp.where(lane == _NPTS, gbg,
                       jnp.zeros((_K, 128), jnp.int32))
    sc_mat = jnp.zeros((_K, 128), jnp.float32)

    for tt in range(_NPTS):
        m2 = jnp.max(cand, axis=2)
        m = jnp.max(m2, axis=1)[:, None, None]
        g3 = jnp.where(cand == m, gidx, big)
        g2 = jnp.min(g3, axis=2)
        g = jnp.min(g2, axis=1)[:, None]
        sc_mat = jnp.where(lane == tt, m[:, :, 0], sc_mat)
        ix_mat = jnp.where(lane == tt, g, ix_mat)
        cand = jnp.where(gidx == g[:, :, None], neg, cand)

    sc_ref[...] = sc_mat
    ix_ref[...] = ix_mat


@functools.partial(jax.jit, static_argnames=("interpret",))
def _run(target2, reference_feats, interpret=False):
    sim = pl.pallas_call(
        _sim_body,
        out_shape=jax.ShapeDtypeStruct((_K, _H * _H), jnp.float32),
        interpret=interpret,
    )(target2, reference_feats)

    sim2 = sim.reshape(_K * _H, _H)
    simt = sim.reshape(_K, _H, _H).transpose(0, 2, 1).reshape(_K * _H, _H)

    sc, ix = pl.pallas_call(
        _sel_body,
        out_shape=[
            jax.ShapeDtypeStruct((_K, 128), jnp.float32),
            jax.ShapeDtypeStruct((_K, 128), jnp.int32),
        ],
        scratch_shapes=[
            pltpu.VMEM((_K * _H, _OH), jnp.float32),
            pltpu.VMEM((_K * 16, _OH), jnp.float32),
            pltpu.VMEM((_K, _OH), jnp.float32),
            pltpu.VMEM((_K, _OH), jnp.float32),
        ],
        interpret=interpret,
    )(sim2, simt, jnp.asarray(_WX), jnp.asarray(_WXET), jnp.asarray(_WY))
    return sc, ix


def kernel(image_embeddings, reference_feats, orig_h, orig_w):
    target2 = image_embeddings.reshape(_C, _H * _H)
    sc, ix = _run(target2, reference_feats)
    scores = sc[:, :_NPTS]
    idx = ix[:, :_NPTS]
    xs = (idx % orig_w).astype(jnp.float32)
    ys = ((idx % (orig_h * orig_w)) // orig_w).astype(jnp.float32)
    points_scores = jnp.stack([xs, ys, scores], axis=-1)
    bgi = ix[:, _NPTS:_NPTS + 1]
    bg_x = (bgi % orig_w).astype(jnp.float32)
    bg_y = ((bgi % (orig_h * orig_w)) // orig_w).astype(jnp.float32)
    bg_coords = jnp.stack([bg_x, bg_y], axis=-1)
    return points_scores, bg_coords
